# baseline (speedup 1.0000x reference)
import functools

import jax
import jax.numpy as jnp
from jax import lax
from jax.experimental import pallas as pl
from jax.experimental.pallas import tpu as pltpu
from jax.experimental.pallas import tpu_sc as plsc

BATCH = 4096
N_FIELDS = 26
EMB = 128
TOT = BATCH * N_FIELDS
NW = 32
PER_W = TOT // NW
CH = 128
NCH = PER_W // CH
RING = 7
DEPTH = 5

_mesh = plsc.VectorSubcoreMesh(core_axis_name="c", subcore_axis_name="s")


@functools.partial(
    pl.kernel,
    mesh=_mesh,
    out_type=jax.ShapeDtypeStruct((TOT, EMB), jnp.float32),
    scratch_types=[
        pltpu.VMEM((NCH, CH), jnp.int32),
        pltpu.VMEM((RING, CH, EMB), jnp.float32),
        pltpu.SemaphoreType.DMA((RING,)),
        pltpu.SemaphoreType.DMA((RING,)),
    ],
)
def _gather(table_hbm, idx_hbm, out_hbm, idx_v, rows_v, gsems, wsems):
    wid = lax.axis_index("s") * 2 + lax.axis_index("c")
    base = wid * PER_W
    pltpu.sync_copy(idx_hbm.at[wid], idx_v)

    def wait_gather(r):
        pltpu.make_async_copy(
            table_hbm.at[pl.ds(0, CH)], rows_v.at[r], gsems.at[r]
        ).wait()

    def wait_write(r):
        pltpu.make_async_copy(
            table_hbm.at[pl.ds(0, CH)], rows_v.at[r], wsems.at[r]
        ).wait()

    def start_gather(j, r):
        pltpu.async_copy(table_hbm.at[idx_v.at[j]], rows_v.at[r], gsems.at[r])

    def start_write(j, r):
        pltpu.async_copy(
            rows_v.at[r], out_hbm.at[pl.ds(base + j * CH, CH)], wsems.at[r]
        )

    for j in range(DEPTH):
        start_gather(j, j % RING)
    for j in range(NCH):
        r = j % RING
        wait_gather(r)
        start_write(j, r)
        nf = j + DEPTH
        if nf < NCH:
            q = nf % RING
            if nf >= RING:
                wait_write(q)
            start_gather(nf, q)
    for r in range(RING):
        wait_write(r)


def kernel(inputs, embedding):
    idx = inputs.astype(jnp.int32).T.reshape(NW, NCH, CH)
    out = _gather(embedding, idx)
    return out.reshape(N_FIELDS, BATCH, EMB).transpose(1, 0, 2)


# speedup vs baseline: 1.0087x; 1.0087x over previous
import functools

import jax
import jax.numpy as jnp
from jax import lax
from jax.experimental import pallas as pl
from jax.experimental.pallas import tpu as pltpu
from jax.experimental.pallas import tpu_sc as plsc

BATCH = 4096
N_FIELDS = 26
EMB = 128
TOT = BATCH * N_FIELDS
NW = 32
PER_W = TOT // NW
CH = 104
NCH = PER_W // CH
RING = 8
DEPTH = 4

_mesh = plsc.VectorSubcoreMesh(core_axis_name="c", subcore_axis_name="s")


@functools.partial(
    pl.kernel,
    mesh=_mesh,
    out_type=jax.ShapeDtypeStruct((TOT, EMB), jnp.float32),
    scratch_types=[
        pltpu.VMEM((NCH, CH), jnp.int32),
        pltpu.VMEM((RING, CH, EMB), jnp.float32),
        pltpu.SemaphoreType.DMA((RING,)),
        pltpu.SemaphoreType.DMA((RING,)),
    ],
)
def _gather(table_hbm, idx_hbm, out_hbm, idx_v, rows_v, gsems, wsems):
    wid = lax.axis_index("s") * 2 + lax.axis_index("c")
    base = wid * PER_W
    pltpu.sync_copy(idx_hbm.at[wid], idx_v)

    def wait_gather(r):
        pltpu.make_async_copy(
            table_hbm.at[pl.ds(0, CH)], rows_v.at[r], gsems.at[r]
        ).wait()

    def wait_write(r):
        pltpu.make_async_copy(
            table_hbm.at[pl.ds(0, CH)], rows_v.at[r], wsems.at[r]
        ).wait()

    def start_gather(j, r):
        pltpu.async_copy(table_hbm.at[idx_v.at[j]], rows_v.at[r], gsems.at[r])

    def start_write(j, r):
        pltpu.async_copy(
            rows_v.at[r], out_hbm.at[pl.ds(base + j * CH, CH)], wsems.at[r]
        )

    for j in range(DEPTH):
        start_gather(j, j)
    for j in range(DEPTH):
        wait_gather(j)
        start_write(j, j)
        start_gather(j + DEPTH, j + DEPTH)

    def group(g, carry):
        for b in range(RING):
            j = DEPTH + g * RING + b
            rr = (DEPTH + b) % RING
            q = (rr + DEPTH) % RING
            wait_gather(rr)
            start_write(j, rr)
            wait_write(q)
            start_gather(j + DEPTH, q)
        return carry

    lax.fori_loop(0, (NCH - 2 * DEPTH) // RING, group, 0)

    for b in range(DEPTH):
        j = NCH - DEPTH + b
        rr = j % RING
        wait_gather(rr)
        start_write(j, rr)
    for r in range(RING):
        wait_write(r)


def kernel(inputs, embedding):
    idx = inputs.astype(jnp.int32).T.reshape(NW, NCH, CH)
    out = _gather(embedding, idx)
    return out.reshape(N_FIELDS, BATCH, EMB).transpose(1, 0, 2)
